# baseline (speedup 1.0000x reference)
import jax
import jax.numpy as jnp
from jax import lax
from jax.experimental import pallas as pl
from jax.experimental.pallas import tpu as pltpu
from jax.experimental.pallas import tpu_sc as plsc

B = 16384
D = 64
NC = 2
NS = 16
NW = NC * NS
BPW = B // NW
L = 16
NROW2 = 500000


def _bias_body(idx3_hbm, ub_hbm, ib_hbm, bsum_hbm, idx_v, bv, sem):
    wid = lax.axis_index("s") * NC + lax.axis_index("c")
    pltpu.sync_copy(idx3_hbm.at[wid], idx_v)
    descs = []
    for c in range(4):
        descs.append(pltpu.async_copy(ub_hbm.at[idx_v.at[c]], bv.at[c], sem))
        descs.append(pltpu.async_copy(ib_hbm.at[idx_v.at[c + 4]], bv.at[c + 4], sem))
    for d_ in descs:
        d_.wait()
    for c in range(4):
        for j in range(128 // L):
            s = pl.ds(j * L, L)
            bv[c, s] = bv[c, s] + bv[c + 4, s]
    pltpu.sync_copy(bv, bsum_hbm.at[wid])


def _dot_body(uemb_hbm, iemb_hbm, idx3_hbm, bsum_hbm, out_hbm,
              idx_v, bs_v, out_v,
              iu0, iu1, iu2, iu3, ii0, ii1, ii2, ii3,
              ue_v, ie_v, sem):
    wid = lax.axis_index("s") * NC + lax.axis_index("c")
    pltpu.sync_copy(idx3_hbm.at[wid], idx_v)
    pltpu.sync_copy(bsum_hbm.at[wid], bs_v)

    iu = [iu0, iu1, iu2, iu3]
    ii = [ii0, ii1, ii2, ii3]
    for c in range(4):
        for j in range(128 // L):
            s = pl.ds(j * L, L)
            iu[c][s] = lax.shift_right_logical(idx_v[c, s], 1)
            ii[c][s] = lax.shift_right_logical(idx_v[c + 4, s], 1)

    lanes = lax.iota(jnp.int32, L)
    for h in range(2):
        descs = []
        for cc in range(2):
            c = 2 * h + cc
            descs.append(pltpu.async_copy(
                uemb_hbm.at[iu[c]], ue_v.at[pl.ds(cc * 128, 128)], sem))
            descs.append(pltpu.async_copy(
                iemb_hbm.at[ii[c]], ie_v.at[pl.ds(cc * 128, 128)], sem))
        for d_ in descs:
            d_.wait()

        def grp_body(j, _, h=h):
            l0 = j * L
            cl = lax.shift_right_logical(l0, 7)
            lm = lax.bitwise_and(l0, 127)
            cu = 2 * h + cl
            pu = lax.bitwise_and(idx_v[cu, pl.ds(lm, L)], 1) * D
            pi = lax.bitwise_and(idx_v[cu + 4, pl.ds(lm, L)], 1) * D
            vec = jnp.zeros((L,), jnp.float32)
            for t in range(L):
                lr = l0 + t
                ou = pu[t]
                oi = pi[t]
                acc = (ue_v[lr, pl.ds(ou, L)] * ie_v[lr, pl.ds(oi, L)])
                for k in range(1, D // L):
                    acc = acc + (ue_v[lr, pl.ds(ou + k * L, L)]
                                 * ie_v[lr, pl.ds(oi + k * L, L)])
                vec = jnp.where(lanes == t, jnp.sum(acc), vec)
            r = vec + bs_v[cu, pl.ds(lm, L)]
            out_v[cu, pl.ds(lm, L)] = 5.0 / (1.0 + jnp.exp(-r))
            return 0

        lax.fori_loop(0, 256 // L, grp_body, 0)

    pltpu.sync_copy(out_v, out_hbm.at[wid])


def kernel(x_batch, user_emb, item_emb, user_bias, item_bias):
    uemb2 = user_emb.reshape(NROW2, 128)
    iemb2 = item_emb.reshape(NROW2, 128)
    ub = user_bias.reshape(-1)
    ib = item_bias.reshape(-1)
    u4 = x_batch[:, 0].reshape(NW, 4, 128)
    i4 = x_batch[:, 1].reshape(NW, 4, 128)
    idx3 = jnp.concatenate([u4, i4], axis=1)

    mesh = plsc.VectorSubcoreMesh(core_axis_name="c", subcore_axis_name="s")

    bias_k = pl.kernel(
        _bias_body,
        out_type=jax.ShapeDtypeStruct((NW, 8, 128), jnp.float32),
        mesh=mesh,
        compiler_params=pltpu.CompilerParams(
            needs_layout_passes=False, use_tc_tiling_on_sc=False
        ),
        scratch_types=[
            pltpu.VMEM((8, 128), jnp.int32),
            pltpu.VMEM((8, 128), jnp.float32),
            pltpu.SemaphoreType.DMA,
        ],
    )
    bsum = bias_k(idx3, ub, ib)

    dot_k = pl.kernel(
        _dot_body,
        out_type=jax.ShapeDtypeStruct((NW, 8, 128), jnp.float32),
        mesh=mesh,
        compiler_params=pltpu.CompilerParams(
            needs_layout_passes=False, use_tc_tiling_on_sc=True
        ),
        scratch_types=[
            pltpu.VMEM((8, 128), jnp.int32),
            pltpu.VMEM((8, 128), jnp.float32),
            pltpu.VMEM((8, 128), jnp.float32),
            pltpu.VMEM((128,), jnp.int32),
            pltpu.VMEM((128,), jnp.int32),
            pltpu.VMEM((128,), jnp.int32),
            pltpu.VMEM((128,), jnp.int32),
            pltpu.VMEM((128,), jnp.int32),
            pltpu.VMEM((128,), jnp.int32),
            pltpu.VMEM((128,), jnp.int32),
            pltpu.VMEM((128,), jnp.int32),
            pltpu.VMEM((256, 128), jnp.float32),
            pltpu.VMEM((256, 128), jnp.float32),
            pltpu.SemaphoreType.DMA,
        ],
    )
    out3 = dot_k(uemb2, iemb2, idx3, bsum)
    return out3[:, :4, :].reshape(B)
